# baseline (speedup 1.0000x reference)
import functools

import jax
import jax.numpy as jnp
from jax import lax
from jax.experimental import pallas as pl
from jax.experimental.pallas import tpu as pltpu
from jax.experimental.pallas import tpu_sc as plsc

N_NODES = 10000
N_FACES = 20000
M = 10000
K = 32
C = 128

P1 = 32
SPAN1 = 640
NT1 = SPAN1 // P1
CH = 80
R = M * K
NW = 32
RPT = R // NW
NT2 = RPT // CH

OFF_F = 0
OFF_B = OFF_F + 3 * N_FACES
OFF_I = OFF_B + 3 * M
OFF_XG = OFF_I + M
CAT_LEN = OFF_XG + R


def _sc_expand(xf, cat):
    mesh = plsc.VectorSubcoreMesh(core_axis_name="c", subcore_axis_name="s")

    @functools.partial(
        pl.kernel,
        out_type=jax.ShapeDtypeStruct((R, C), jnp.float32),
        mesh=mesh,
        scratch_types=[
            pltpu.VMEM_SHARED((M, C), jnp.float32),
            pltpu.SemaphoreType.DMA,
            pltpu.SemaphoreType.DMA,
            pltpu.SemaphoreType.DMA,
            pltpu.SemaphoreType.DMA,
            pltpu.SemaphoreType.DMA,
        ],
    )
    def body(xf_h, cat_h, out_h,
             nsh, semv, sembx, semby, semg, semo):
        cid = lax.axis_index("c")
        sid = lax.axis_index("s")
        wid = sid * 2 + cid

        span_start = jnp.minimum(sid * SPAN1, M - SPAN1)

        def phase1(i_all, vidx2, bidx2, vid2, bar2, rows2, n2):
            pltpu.sync_copy(cat_h.at[pl.ds(OFF_I + span_start, SPAN1)], i_all)
            sems = (sembx, semby)

            def cbase(t):
                return jnp.minimum(span_start + t * P1, M - P1)

            def s1(t, z):
                cb = cbase(t)
                off = cb - span_start
                for g in range(P1 // 16):
                    gsl = pl.ds(g * 16, 16)
                    iv3 = i_all[pl.ds(off + g * 16, 16)] * 3
                    pb3 = OFF_B + (cb + g * 16 + lax.iota(jnp.int32, 16)) * 3
                    for j in range(3):
                        vidx2[z, j, gsl] = iv3 + j
                        bidx2[z, j, gsl] = pb3 + j
                return [pltpu.async_copy(cat_h.at[vidx2.at[z, j]],
                                         vid2.at[z, j], semv)
                        for j in range(3)]

            def s2(dv, z):
                for cp in dv:
                    cp.wait()
                for j in range(3):
                    pltpu.async_copy(xf_h.at[vid2.at[z, j]],
                                     rows2.at[z, j], sems[z])
                for j in range(3):
                    pltpu.async_copy(cat_h.at[bidx2.at[z, j]],
                                     bar2.at[z, j], sems[z])

            def s3(z):
                for j in range(3):
                    pltpu.make_async_copy(xf_h.at[pl.ds(0, P1)],
                                          rows2.at[z, j], sems[z]).wait()
                for j in range(3):
                    pltpu.make_async_copy(cat_h.at[pl.ds(0, P1)],
                                          bar2.at[z, j], sems[z]).wait()

            def s4(t, z):
                def fgroup(g, _):
                    gsl = pl.ds(g * 16, 16)
                    bv = [bar2[z, j, gsl] for j in range(3)]
                    for l in range(16):
                        p = g * 16 + l
                        b0, b1, b2 = (
                            lax.bitcast_convert_type(bv[0][l], jnp.float32),
                            lax.bitcast_convert_type(bv[1][l], jnp.float32),
                            lax.bitcast_convert_type(bv[2][l], jnp.float32))
                        for cc in range(C // 16):
                            sl = pl.ds(cc * 16, 16)
                            n2[z, p, sl] = (rows2[z, 0, p, sl] * b0
                                            + rows2[z, 1, p, sl] * b1
                                            + rows2[z, 2, p, sl] * b2)
                    return 0
                lax.fori_loop(0, P1 // 16, fgroup, 0)
                pltpu.sync_copy(n2.at[z], nsh.at[pl.ds(cbase(t), P1)])

            s2(s1(0, 0), 0)

            def kk_body(kk, _):
                a = 2 * kk
                s2(s1(a + 1, 1), 1)
                s3(0)
                s4(a, 0)

                @pl.when(kk < NT1 // 2 - 1)
                def _():
                    s2(s1(a + 2, 0), 0)
                s3(1)
                s4(a + 1, 1)
                return 0
            lax.fori_loop(0, NT1 // 2, kk_body, 0)

        pl.run_scoped(
            phase1,
            pltpu.VMEM((SPAN1,), jnp.int32),
            pltpu.VMEM((2, 3, P1), jnp.int32),
            pltpu.VMEM((2, 3, P1), jnp.int32),
            pltpu.VMEM((2, 3, P1), jnp.int32),
            pltpu.VMEM((2, 3, P1), jnp.int32),
            pltpu.VMEM((2, 3, P1, C), jnp.float32),
            pltpu.VMEM((2, P1, C), jnp.float32),
        )

        plsc.subcore_barrier()

        def phase2(gidxa, gx, gy):
            rbase = wid * RPT
            pltpu.sync_copy(cat_h.at[pl.ds(OFF_XG + rbase, RPT)], gidxa)

            def start_gather(t, buf):
                pltpu.async_copy(nsh.at[gidxa.at[pl.ds(t * CH, CH)]],
                                 buf, semg)

            def drain_gather(buf):
                pltpu.make_async_copy(out_h.at[pl.ds(0, CH)], buf,
                                      semg).wait()

            def start_out(t, buf):
                pltpu.async_copy(buf, out_h.at[pl.ds(rbase + t * CH, CH)],
                                 semo)

            def drain_out(buf):
                pltpu.make_async_copy(out_h.at[pl.ds(0, CH)], buf,
                                      semo).wait()

            start_gather(0, gx)

            def kk_body(kk, _):
                t0 = 2 * kk
                drain_gather(gx)

                @pl.when(kk > 0)
                def _():
                    drain_out(gy)
                start_out(t0, gx)
                start_gather(t0 + 1, gy)
                drain_gather(gy)
                drain_out(gx)
                start_out(t0 + 1, gy)
                start_gather(t0 + 2, gx)
                return 0
            lax.fori_loop(0, (NT2 - 1) // 2, kk_body, 0)

            drain_gather(gx)
            drain_out(gy)
            start_out(NT2 - 1, gx)
            drain_out(gx)

        pl.run_scoped(
            phase2,
            pltpu.VMEM((RPT,), jnp.int32),
            pltpu.VMEM((CH, C), jnp.float32),
            pltpu.VMEM((CH, C), jnp.float32),
        )

    return body(xf, cat)


def kernel(x_features, x_graph, F, I, bary):
    xf = x_features.reshape(N_NODES, C)
    cat = jnp.concatenate([
        F.astype(jnp.int32).reshape(3 * N_FACES),
        jax.lax.bitcast_convert_type(
            bary.astype(jnp.float32).reshape(3 * M), jnp.int32),
        I.astype(jnp.int32),
        x_graph.astype(jnp.int32).reshape(R),
    ])
    out = _sc_expand(xf, cat)
    return out.reshape(1, M, K, C)


# speedup vs baseline: 1.0778x; 1.0778x over previous
import functools

import jax
import jax.numpy as jnp
from jax import lax
from jax.experimental import pallas as pl
from jax.experimental.pallas import tpu as pltpu
from jax.experimental.pallas import tpu_sc as plsc

N_NODES = 10000
N_FACES = 20000
M = 10000
K = 32
C = 128

P1 = 32
SPAN1 = 640
NT1 = SPAN1 // P1
CH = 80
R = M * K
NW = 32
RPT = R // NW
NT2 = RPT // CH

def _sc_expand(xf, xg_flat, f_flat, i_arr, bar_flat):
    mesh = plsc.VectorSubcoreMesh(core_axis_name="c", subcore_axis_name="s")

    @functools.partial(
        pl.kernel,
        out_type=jax.ShapeDtypeStruct((R, C), jnp.float32),
        mesh=mesh,
        scratch_types=[
            pltpu.VMEM_SHARED((M, C), jnp.float32),
            pltpu.SemaphoreType.DMA,
            pltpu.SemaphoreType.DMA,
            pltpu.SemaphoreType.DMA,
            pltpu.SemaphoreType.DMA,
            pltpu.SemaphoreType.DMA,
        ],
    )
    def body(xf_h, xg_h, f_h, i_h, b_h, out_h,
             nsh, semv, sembx, semby, semg, semo):
        cid = lax.axis_index("c")
        sid = lax.axis_index("s")
        wid = sid * 2 + cid

        span_start = jnp.minimum(sid * SPAN1, M - SPAN1)

        def phase1(i_all, vidx2, bidx2, vid2, bar2, rows2, n2):
            pltpu.sync_copy(i_h.at[pl.ds(span_start, SPAN1)], i_all)
            sems = (sembx, semby)

            def cbase(t):
                return jnp.minimum(span_start + t * P1, M - P1)

            def s1(t, z):
                cb = cbase(t)
                off = cb - span_start
                for g in range(P1 // 16):
                    gsl = pl.ds(g * 16, 16)
                    iv3 = i_all[pl.ds(off + g * 16, 16)] * 3
                    pb3 = (cb + g * 16 + lax.iota(jnp.int32, 16)) * 3
                    for j in range(3):
                        vidx2[z, j, gsl] = iv3 + j
                        bidx2[z, j, gsl] = pb3 + j
                return [pltpu.async_copy(f_h.at[vidx2.at[z, j]],
                                         vid2.at[z, j], semv)
                        for j in range(3)]

            def s2(dv, z):
                for cp in dv:
                    cp.wait()
                for j in range(3):
                    pltpu.async_copy(xf_h.at[vid2.at[z, j]],
                                     rows2.at[z, j], sems[z])
                for j in range(3):
                    pltpu.async_copy(b_h.at[bidx2.at[z, j]],
                                     bar2.at[z, j], sems[z])

            def s3(z):
                for j in range(3):
                    pltpu.make_async_copy(xf_h.at[pl.ds(0, P1)],
                                          rows2.at[z, j], sems[z]).wait()
                for j in range(3):
                    pltpu.make_async_copy(b_h.at[pl.ds(0, P1)],
                                          bar2.at[z, j], sems[z]).wait()

            def s4(t, z):
                def fgroup(g, _):
                    gsl = pl.ds(g * 16, 16)
                    bv = [bar2[z, j, gsl] for j in range(3)]
                    for l in range(16):
                        p = g * 16 + l
                        b0, b1, b2 = bv[0][l], bv[1][l], bv[2][l]
                        for cc in range(C // 16):
                            sl = pl.ds(cc * 16, 16)
                            n2[z, p, sl] = (rows2[z, 0, p, sl] * b0
                                            + rows2[z, 1, p, sl] * b1
                                            + rows2[z, 2, p, sl] * b2)
                    return 0
                lax.fori_loop(0, P1 // 16, fgroup, 0)
                pltpu.sync_copy(n2.at[z], nsh.at[pl.ds(cbase(t), P1)])

            s2(s1(0, 0), 0)

            def kk_body(kk, _):
                a = 2 * kk
                s2(s1(a + 1, 1), 1)
                s3(0)
                s4(a, 0)

                @pl.when(kk < NT1 // 2 - 1)
                def _():
                    s2(s1(a + 2, 0), 0)
                s3(1)
                s4(a + 1, 1)
                return 0
            lax.fori_loop(0, NT1 // 2, kk_body, 0)

        pl.run_scoped(
            phase1,
            pltpu.VMEM((SPAN1,), jnp.int32),
            pltpu.VMEM((2, 3, P1), jnp.int32),
            pltpu.VMEM((2, 3, P1), jnp.int32),
            pltpu.VMEM((2, 3, P1), jnp.int32),
            pltpu.VMEM((2, 3, P1), jnp.float32),
            pltpu.VMEM((2, 3, P1, C), jnp.float32),
            pltpu.VMEM((2, P1, C), jnp.float32),
        )

        plsc.subcore_barrier()

        def phase2(gidxa, gx, gy):
            rbase = wid * RPT
            pltpu.sync_copy(xg_h.at[pl.ds(rbase, RPT)], gidxa)

            def start_gather(t, buf):
                pltpu.async_copy(nsh.at[gidxa.at[pl.ds(t * CH, CH)]],
                                 buf, semg)

            def drain_gather(buf):
                pltpu.make_async_copy(out_h.at[pl.ds(0, CH)], buf,
                                      semg).wait()

            def start_out(t, buf):
                pltpu.async_copy(buf, out_h.at[pl.ds(rbase + t * CH, CH)],
                                 semo)

            def drain_out(buf):
                pltpu.make_async_copy(out_h.at[pl.ds(0, CH)], buf,
                                      semo).wait()

            start_gather(0, gx)

            def kk_body(kk, _):
                t0 = 2 * kk
                drain_gather(gx)

                @pl.when(kk > 0)
                def _():
                    drain_out(gy)
                start_out(t0, gx)
                start_gather(t0 + 1, gy)
                drain_gather(gy)
                drain_out(gx)
                start_out(t0 + 1, gy)

                @pl.when(t0 + 2 < NT2)
                def _():
                    start_gather(t0 + 2, gx)
                return 0
            lax.fori_loop(0, NT2 // 2, kk_body, 0)
            if NT2 % 2:
                drain_gather(gx)
                drain_out(gy)
                start_out(NT2 - 1, gx)
                drain_out(gx)
            else:
                drain_out(gy)

        pl.run_scoped(
            phase2,
            pltpu.VMEM((RPT,), jnp.int32),
            pltpu.VMEM((CH, C), jnp.float32),
            pltpu.VMEM((CH, C), jnp.float32),
        )

    return body(xf, xg_flat, f_flat, i_arr, bar_flat)


def kernel(x_features, x_graph, F, I, bary):
    xf = x_features.reshape(N_NODES, C)
    xg_flat = x_graph.reshape(R)
    f_flat = F.reshape(3 * N_FACES)
    bar_flat = bary.reshape(3 * M)
    out = _sc_expand(xf, xg_flat, f_flat, I, bar_flat)
    return out.reshape(1, M, K, C)
